# baseline (speedup 1.0000x reference)
import functools

import jax
import jax.numpy as jnp
from jax import lax
from jax.experimental import pallas as pl
from jax.experimental.pallas import tpu as pltpu
from jax.experimental.pallas import tpu_sc as plsc

B = 10
N = 1000
NP = 1024
K = 16
C = 32
NCLS = 40
TOPK_CHUNK = 128
E = B * NP * K
_BIG = 3.0e38


def _tc1_body(posP, posT, wa1, wp1, b1a, w1b, b1b, wh2, wp2, b2a,
              idxO, aO, bO, aS, bS, h1S):
    g = pl.program_id(0)
    p = posP[0]
    pT = posT[0]
    col_row = lax.broadcasted_iota(jnp.int32, (1, NP), 1)
    sq_row = (jnp.sum(pT * pT, axis=0, keepdims=True)
              + jnp.where(col_row >= N, _BIG, 0.0))
    col_i = lax.broadcasted_iota(jnp.int32, (TOPK_CHUNK, NP), 1)

    aS[:, :] = jnp.dot(p, wa1[:, :]) + b1a[0:1, :]
    bS[:, :] = jnp.dot(p, wp1[:, :])

    def topk_chunk(c, _):
        base = c * TOPK_CHUNK
        pc = posP[0, pl.ds(base, TOPK_CHUNK), :]
        sqc = jnp.sum(pc * pc, axis=1, keepdims=True)
        d = (sqc + sq_row) - 2.0 * jnp.dot(pc, pT)
        bc_ = bS[pl.ds(base, TOPK_CHUNK), :]
        bits = lax.bitcast_convert_type(d, jnp.int32)
        s = bits ^ ((bits >> 31) & jnp.int32(0x7FFFFFFF))
        key = (s & jnp.int32(-1024)) | col_i
        cols = []
        mx = None
        for _k in range(K):
            kmin = jnp.min(key, axis=1, keepdims=True)
            z = key == kmin
            key = jnp.where(z, jnp.int32(0x7FFFFFFF), key)
            cols.append(kmin & jnp.int32(1023))
            zf = z.astype(jnp.float32)
            gk = jnp.dot(zf, aS[:, :])
            msg = jnp.dot(jnp.maximum(gk - bc_, 0.0), w1b[:, :])
            mx = msg if mx is None else jnp.maximum(mx, msg)
        idxO[0, pl.ds(base, TOPK_CHUNK), :] = (
            jnp.concatenate(cols, axis=1) + g * NP)
        h1S[pl.ds(base, TOPK_CHUNK), :] = jnp.maximum(mx + b1b[0:1, :], 0.0)
        return 0

    lax.fori_loop(0, NP // TOPK_CHUNK, topk_chunk, 0, unroll=False)

    p2 = jnp.dot(p, wp2[:, :])
    aO[0, :, 0:C] = jnp.dot(h1S[:, :], wh2[:, :]) + p2 + b2a[0:1, :]
    aO[0, :, C:128] = jnp.zeros((NP, 128 - C), jnp.float32)
    bO[0] = p2


def _tc2_body(edges, b2t, w2b, b2b, wc, bc, out):
    e3 = edges[0, :, 0:C].reshape(NP, K, C)
    pre = e3 - b2t[0][:, None, :]
    act = jnp.maximum(pre, 0.0).reshape(NP * K, C)
    msg = (jnp.dot(act, w2b[:, :]) + b2b[0:1, :]).reshape(NP, K, C)
    h2 = jnp.maximum(jnp.max(msg, axis=1), 0.0)
    row_i = lax.broadcasted_iota(jnp.int32, (NP, C), 0)
    hm = jnp.where(row_i >= N, 0.0, h2)
    gmax = jnp.max(hm, axis=0, keepdims=True)
    out[0] = jnp.dot(gmax, wc[:, :]) + bc[0:1, :]


_SC_CHUNK = 128


def _sc_gather(table, idx):
    info = plsc.get_sparse_core_info()
    nw = info.num_cores * info.num_subcores
    b_per_w = E // nw
    n_chunks = b_per_w // _SC_CHUNK
    mesh = plsc.VectorSubcoreMesh(core_axis_name="c", subcore_axis_name="s")

    @functools.partial(
        pl.kernel, mesh=mesh,
        out_type=jax.ShapeDtypeStruct((E, 128), jnp.float32),
        scratch_types=[
            pltpu.VMEM((_SC_CHUNK,), jnp.int32),
            pltpu.VMEM((_SC_CHUNK, 128), jnp.float32),
            pltpu.SemaphoreType.DMA,
        ],
    )
    def k(table_hbm, idx_hbm, out_hbm, idx_v, rows_v, sem):
        wid = lax.axis_index("s") * info.num_cores + lax.axis_index("c")
        base_w = wid * b_per_w

        def chunk(c, _):
            base = base_w + c * _SC_CHUNK
            pltpu.sync_copy(idx_hbm.at[pl.ds(base, _SC_CHUNK)], idx_v)
            pltpu.async_copy(table_hbm.at[idx_v], rows_v, sem).wait()
            pltpu.sync_copy(rows_v, out_hbm.at[pl.ds(base, _SC_CHUNK)])
            return 0

        lax.fori_loop(0, n_chunks, chunk, 0, unroll=False)

    return k(table, idx)


def _full(shape):
    nd = len(shape)
    return pl.BlockSpec(shape, lambda g, _n=nd: (0,) * _n)


def _padrow(x, rows=8):
    return jnp.zeros((rows, x.shape[-1]), x.dtype).at[: x.shape[0]].set(x)


@functools.partial(jax.jit, static_argnames=())
def kernel(pos, batch, W1a, b1a, W1b, b1b, W2a, b2a, W2b, b2b, Wc, bc):
    P = pos.reshape(B, N, 3)
    posP = jnp.zeros((B, NP, 8), jnp.float32).at[:, :N, :3].set(P)
    posT = posP.transpose(0, 2, 1)
    wa1 = _padrow(W1a[0:3] + W1a[3:6])
    wp1 = _padrow(W1a[3:6])
    wh2 = W2a[0:C]
    wp2 = _padrow(W2a[C:C + 3])

    idxG, a2, b2t = pl.pallas_call(
        _tc1_body,
        grid=(B,),
        in_specs=[
            pl.BlockSpec((1, NP, 8), lambda g: (g, 0, 0)),
            pl.BlockSpec((1, 8, NP), lambda g: (g, 0, 0)),
            _full((8, C)), _full((8, C)), _full((8, C)), _full((C, C)),
            _full((8, C)), _full((C, C)), _full((8, C)), _full((8, C)),
        ],
        out_specs=[
            pl.BlockSpec((1, NP, K), lambda g: (g, 0, 0)),
            pl.BlockSpec((1, NP, 128), lambda g: (g, 0, 0)),
            pl.BlockSpec((1, NP, C), lambda g: (g, 0, 0)),
        ],
        out_shape=[
            jax.ShapeDtypeStruct((B, NP, K), jnp.int32),
            jax.ShapeDtypeStruct((B, NP, 128), jnp.float32),
            jax.ShapeDtypeStruct((B, NP, C), jnp.float32),
        ],
        scratch_shapes=[
            pltpu.VMEM((NP, C), jnp.float32),
            pltpu.VMEM((NP, C), jnp.float32),
            pltpu.VMEM((NP, C), jnp.float32),
        ],
    )(posP, posT, wa1, wp1, _padrow(b1a[None, :]), W1b,
      _padrow(b1b[None, :]), wh2, wp2, _padrow(b2a[None, :]))

    edges = _sc_gather(a2.reshape(B * NP, 128), idxG.reshape(E))

    out = pl.pallas_call(
        _tc2_body,
        grid=(B,),
        in_specs=[
            pl.BlockSpec((1, NP * K, 128), lambda g: (g, 0, 0)),
            pl.BlockSpec((1, NP, C), lambda g: (g, 0, 0)),
            _full((C, C)), _full((8, C)), _full((C, NCLS)),
            _full((8, NCLS)),
        ],
        out_specs=pl.BlockSpec((1, 1, NCLS), lambda g: (g, 0, 0)),
        out_shape=jax.ShapeDtypeStruct((B, 1, NCLS), jnp.float32),
    )(edges.reshape(B, NP * K, 128), b2t, W2b, _padrow(b2b[None, :]), Wc,
      _padrow(bc[None, :]))
    return out.reshape(B, NCLS)


# speedup vs baseline: 1.0747x; 1.0747x over previous
import functools

import jax
import jax.numpy as jnp
from jax import lax
from jax.experimental import pallas as pl
from jax.experimental.pallas import tpu as pltpu
from jax.experimental.pallas import tpu_sc as plsc

B = 10
N = 1000
NP = 1024
K = 16
C = 32
NCLS = 40
TOPK_CHUNK = 128
E = B * NP * K
_BIG = 3.0e38


def _tc1_body(posP, posT, wa1, wp1, b1a, w1b, b1b, wh2, wp2, b2a,
              idxO, aO, bO, aS, bS, h1S):
    g = pl.program_id(0)
    p = posP[0]
    pT = posT[0]
    col_row = lax.broadcasted_iota(jnp.int32, (1, NP), 1)
    sq_row = (jnp.sum(pT * pT, axis=0, keepdims=True)
              + jnp.where(col_row >= N, _BIG, 0.0))
    col_i = lax.broadcasted_iota(jnp.int32, (TOPK_CHUNK, NP), 1)

    aS[:, :] = jnp.dot(p, wa1[:, :]) + b1a[0:1, :]
    bS[:, :] = jnp.dot(p, wp1[:, :])

    def topk_chunk(c, _):
        base = c * TOPK_CHUNK
        pc = posP[0, pl.ds(base, TOPK_CHUNK), :]
        sqc = jnp.sum(pc * pc, axis=1, keepdims=True)
        d = (sqc + sq_row) - 2.0 * jnp.dot(pc, pT)
        bc_ = bS[pl.ds(base, TOPK_CHUNK), :]
        bits = lax.bitcast_convert_type(d, jnp.int32)
        s = bits ^ ((bits >> 31) & jnp.int32(0x7FFFFFFF))
        key = (s & jnp.int32(-1024)) | col_i
        cols = []
        mx = None
        for _k in range(K):
            kmin = jnp.min(key, axis=1, keepdims=True)
            z = key == kmin
            key = jnp.where(z, jnp.int32(0x7FFFFFFF), key)
            cols.append(kmin & jnp.int32(1023))
            zf = z.astype(jnp.float32)
            gk = jnp.dot(zf, aS[:, :])
            msg = jnp.dot(jnp.maximum(gk - bc_, 0.0), w1b[:, :])
            mx = msg if mx is None else jnp.maximum(mx, msg)
        idxO[0, pl.ds(base, TOPK_CHUNK), :] = (
            jnp.concatenate(cols, axis=1) + g * NP)
        h1S[pl.ds(base, TOPK_CHUNK), :] = jnp.maximum(mx + b1b[0:1, :], 0.0)
        return 0

    lax.fori_loop(0, NP // TOPK_CHUNK, topk_chunk, 0, unroll=False)

    p2 = jnp.dot(p, wp2[:, :])
    aO[0, :, 0:C] = jnp.dot(h1S[:, :], wh2[:, :]) + p2 + b2a[0:1, :]
    aO[0, :, C:128] = jnp.zeros((NP, 128 - C), jnp.float32)
    bO[0] = p2


def _tc2_body(edges, b2t, w2b, b2b, wc, bc, out):
    e3 = edges[0, :, 0:C].reshape(NP, K, C)
    pre = e3 - b2t[0][:, None, :]
    act = jnp.maximum(pre, 0.0).reshape(NP * K, C)
    msg = (jnp.dot(act, w2b[:, :]) + b2b[0:1, :]).reshape(NP, K, C)
    h2 = jnp.maximum(jnp.max(msg, axis=1), 0.0)
    row_i = lax.broadcasted_iota(jnp.int32, (NP, C), 0)
    hm = jnp.where(row_i >= N, 0.0, h2)
    gmax = jnp.max(hm, axis=0, keepdims=True)
    out[0] = jnp.dot(gmax, wc[:, :]) + bc[0:1, :]


_SC_CHUNK = 128


def _sc_gather(table, idx):
    info = plsc.get_sparse_core_info()
    nw = info.num_cores * info.num_subcores
    b_per_w = E // nw
    n_chunks = b_per_w // _SC_CHUNK
    mesh = plsc.VectorSubcoreMesh(core_axis_name="c", subcore_axis_name="s")

    @functools.partial(
        pl.kernel, mesh=mesh,
        out_type=jax.ShapeDtypeStruct((E, 128), jnp.float32),
        scratch_types=[
            pltpu.VMEM((_SC_CHUNK,), jnp.int32),
            pltpu.VMEM((_SC_CHUNK,), jnp.int32),
            pltpu.VMEM((_SC_CHUNK, 128), jnp.float32),
            pltpu.VMEM((_SC_CHUNK, 128), jnp.float32),
            pltpu.SemaphoreType.DMA,
            pltpu.SemaphoreType.DMA,
        ],
    )
    def k(table_hbm, idx_hbm, out_hbm, idx0, idx1, rows0, rows1, s0, s1):
        wid = lax.axis_index("s") * info.num_cores + lax.axis_index("c")
        base_w = wid * b_per_w

        def chunk2(c, _):
            b0 = base_w + (2 * c) * _SC_CHUNK
            b1 = b0 + _SC_CHUNK
            pltpu.sync_copy(idx_hbm.at[pl.ds(b0, _SC_CHUNK)], idx0)
            cp0 = pltpu.async_copy(table_hbm.at[idx0], rows0, s0)
            pltpu.sync_copy(idx_hbm.at[pl.ds(b1, _SC_CHUNK)], idx1)
            cp1 = pltpu.async_copy(table_hbm.at[idx1], rows1, s1)
            cp0.wait()
            pltpu.sync_copy(rows0, out_hbm.at[pl.ds(b0, _SC_CHUNK)])
            cp1.wait()
            pltpu.sync_copy(rows1, out_hbm.at[pl.ds(b1, _SC_CHUNK)])
            return 0

        lax.fori_loop(0, n_chunks // 2, chunk2, 0, unroll=False)

    return k(table, idx)


def _full(shape):
    nd = len(shape)
    return pl.BlockSpec(shape, lambda g, _n=nd: (0,) * _n)


def _padrow(x, rows=8):
    return jnp.zeros((rows, x.shape[-1]), x.dtype).at[: x.shape[0]].set(x)


@functools.partial(jax.jit, static_argnames=())
def kernel(pos, batch, W1a, b1a, W1b, b1b, W2a, b2a, W2b, b2b, Wc, bc):
    P = pos.reshape(B, N, 3)
    posP = jnp.zeros((B, NP, 8), jnp.float32).at[:, :N, :3].set(P)
    posT = posP.transpose(0, 2, 1)
    wa1 = _padrow(W1a[0:3] + W1a[3:6])
    wp1 = _padrow(W1a[3:6])
    wh2 = W2a[0:C]
    wp2 = _padrow(W2a[C:C + 3])

    idxG, a2, b2t = pl.pallas_call(
        _tc1_body,
        grid=(B,),
        in_specs=[
            pl.BlockSpec((1, NP, 8), lambda g: (g, 0, 0)),
            pl.BlockSpec((1, 8, NP), lambda g: (g, 0, 0)),
            _full((8, C)), _full((8, C)), _full((8, C)), _full((C, C)),
            _full((8, C)), _full((C, C)), _full((8, C)), _full((8, C)),
        ],
        out_specs=[
            pl.BlockSpec((1, NP, K), lambda g: (g, 0, 0)),
            pl.BlockSpec((1, NP, 128), lambda g: (g, 0, 0)),
            pl.BlockSpec((1, NP, C), lambda g: (g, 0, 0)),
        ],
        out_shape=[
            jax.ShapeDtypeStruct((B, NP, K), jnp.int32),
            jax.ShapeDtypeStruct((B, NP, 128), jnp.float32),
            jax.ShapeDtypeStruct((B, NP, C), jnp.float32),
        ],
        scratch_shapes=[
            pltpu.VMEM((NP, C), jnp.float32),
            pltpu.VMEM((NP, C), jnp.float32),
            pltpu.VMEM((NP, C), jnp.float32),
        ],
    )(posP, posT, wa1, wp1, _padrow(b1a[None, :]), W1b,
      _padrow(b1b[None, :]), wh2, wp2, _padrow(b2a[None, :]))

    edges = _sc_gather(a2.reshape(B * NP, 128), idxG.reshape(E))

    out = pl.pallas_call(
        _tc2_body,
        grid=(B,),
        in_specs=[
            pl.BlockSpec((1, NP * K, 128), lambda g: (g, 0, 0)),
            pl.BlockSpec((1, NP, C), lambda g: (g, 0, 0)),
            _full((C, C)), _full((8, C)), _full((C, NCLS)),
            _full((8, NCLS)),
        ],
        out_specs=pl.BlockSpec((1, 1, NCLS), lambda g: (g, 0, 0)),
        out_shape=jax.ShapeDtypeStruct((B, 1, NCLS), jnp.float32),
    )(edges.reshape(B, NP * K, 128), b2t, W2b, _padrow(b2b[None, :]), Wc,
      _padrow(bc[None, :]))
    return out.reshape(B, NCLS)


# speedup vs baseline: 1.0946x; 1.0185x over previous
import functools

import jax
import jax.numpy as jnp
from jax import lax
from jax.experimental import pallas as pl
from jax.experimental.pallas import tpu as pltpu
from jax.experimental.pallas import tpu_sc as plsc

B = 10
N = 1000
NP = 1024
K = 16
C = 32
NCLS = 40
TOPK_CHUNK = 128
E = B * NP * K
_BIG = 3.0e38


def _tc1_body(posP, posT, wa1, wp1, b1a, w1b, b1b, wh2, wp2, b2a,
              idxO, aO, bO, aS, bS, h1S):
    g = pl.program_id(0)
    p = posP[0]
    pT = posT[0]
    col_row = lax.broadcasted_iota(jnp.int32, (1, NP), 1)
    sq_row = (jnp.sum(pT * pT, axis=0, keepdims=True)
              + jnp.where(col_row >= N, _BIG, 0.0))
    col_i = lax.broadcasted_iota(jnp.int32, (TOPK_CHUNK, NP), 1)

    aS[:, :] = jnp.dot(p, wa1[:, :]) + b1a[0:1, :]
    bS[:, :] = jnp.dot(p, wp1[:, :])

    def topk_chunk(c, _):
        base = c * TOPK_CHUNK
        pc = posP[0, pl.ds(base, TOPK_CHUNK), :]
        sqc = jnp.sum(pc * pc, axis=1, keepdims=True)
        d = (sqc + sq_row) - 2.0 * jnp.dot(pc, pT)
        bc_ = bS[pl.ds(base, TOPK_CHUNK), :]
        bits = lax.bitcast_convert_type(d, jnp.int32)
        s = bits ^ ((bits >> 31) & jnp.int32(0x7FFFFFFF))
        key = (s & jnp.int32(-1024)) | col_i
        cols = []
        mx = None
        for _k in range(K):
            kmin = jnp.min(key, axis=1, keepdims=True)
            z = key == kmin
            key = jnp.where(z, jnp.int32(0x7FFFFFFF), key)
            cols.append(kmin & jnp.int32(1023))
            zf = z.astype(jnp.float32)
            gk = jnp.dot(zf, aS[:, :])
            msg = jnp.dot(jnp.maximum(gk - bc_, 0.0), w1b[:, :])
            mx = msg if mx is None else jnp.maximum(mx, msg)
        idxO[0, pl.ds(base, TOPK_CHUNK), :] = (
            jnp.concatenate(cols, axis=1) + g * NP)
        h1S[pl.ds(base, TOPK_CHUNK), :] = jnp.maximum(mx + b1b[0:1, :], 0.0)
        return 0

    lax.fori_loop(0, NP // TOPK_CHUNK, topk_chunk, 0, unroll=False)

    p2 = jnp.dot(p, wp2[:, :])
    aO[0, :, 0:C] = jnp.dot(h1S[:, :], wh2[:, :]) + p2 + b2a[0:1, :]
    aO[0, :, C:128] = jnp.zeros((NP, 128 - C), jnp.float32)
    bO[0] = p2


def _tc2_body(edges, b2t, w2b, b2b, wc, bc, out):
    e3 = edges[0, :, 0:C].reshape(NP, K, C)
    pre = e3 - b2t[0][:, None, :]
    act = jnp.maximum(pre, 0.0).reshape(NP * K, C)
    msg = (jnp.dot(act, w2b[:, :]) + b2b[0:1, :]).reshape(NP, K, C)
    h2 = jnp.maximum(jnp.max(msg, axis=1), 0.0)
    row_i = lax.broadcasted_iota(jnp.int32, (NP, C), 0)
    hm = jnp.where(row_i >= N, 0.0, h2)
    gmax = jnp.max(hm, axis=0, keepdims=True)
    out[0] = jnp.dot(gmax, wc[:, :]) + bc[0:1, :]


_SC_CHUNK = 128
_NBUF = 4


def _sc_gather(table, idx):
    info = plsc.get_sparse_core_info()
    nw = info.num_cores * info.num_subcores
    b_per_w = E // nw
    n_chunks = b_per_w // _SC_CHUNK
    mesh = plsc.VectorSubcoreMesh(core_axis_name="c", subcore_axis_name="s")

    @functools.partial(
        pl.kernel, mesh=mesh,
        out_type=jax.ShapeDtypeStruct((E, 128), jnp.float32),
        scratch_types=(
            [pltpu.VMEM((_SC_CHUNK,), jnp.int32)] * _NBUF
            + [pltpu.VMEM((_SC_CHUNK, 128), jnp.float32)] * _NBUF
            + [pltpu.SemaphoreType.DMA] * _NBUF
        ),
    )
    def k(table_hbm, idx_hbm, out_hbm, *bufs):
        idxs = bufs[0:_NBUF]
        rows = bufs[_NBUF:2 * _NBUF]
        sems = bufs[2 * _NBUF:3 * _NBUF]
        wid = lax.axis_index("s") * info.num_cores + lax.axis_index("c")
        base_w = wid * b_per_w

        def chunkn(c, _):
            cps = []
            for i in range(_NBUF):
                bi = base_w + (_NBUF * c + i) * _SC_CHUNK
                pltpu.sync_copy(idx_hbm.at[pl.ds(bi, _SC_CHUNK)], idxs[i])
                cps.append(pltpu.async_copy(
                    table_hbm.at[idxs[i]], rows[i], sems[i]))
            for i in range(_NBUF):
                bi = base_w + (_NBUF * c + i) * _SC_CHUNK
                cps[i].wait()
                pltpu.sync_copy(rows[i], out_hbm.at[pl.ds(bi, _SC_CHUNK)])
            return 0

        lax.fori_loop(0, n_chunks // _NBUF, chunkn, 0, unroll=False)

    return k(table, idx)


def _full(shape):
    nd = len(shape)
    return pl.BlockSpec(shape, lambda g, _n=nd: (0,) * _n)


def _padrow(x, rows=8):
    return jnp.zeros((rows, x.shape[-1]), x.dtype).at[: x.shape[0]].set(x)


@functools.partial(jax.jit, static_argnames=())
def kernel(pos, batch, W1a, b1a, W1b, b1b, W2a, b2a, W2b, b2b, Wc, bc):
    P = pos.reshape(B, N, 3)
    posP = jnp.zeros((B, NP, 8), jnp.float32).at[:, :N, :3].set(P)
    posT = posP.transpose(0, 2, 1)
    wa1 = _padrow(W1a[0:3] + W1a[3:6])
    wp1 = _padrow(W1a[3:6])
    wh2 = W2a[0:C]
    wp2 = _padrow(W2a[C:C + 3])

    idxG, a2, b2t = pl.pallas_call(
        _tc1_body,
        grid=(B,),
        in_specs=[
            pl.BlockSpec((1, NP, 8), lambda g: (g, 0, 0)),
            pl.BlockSpec((1, 8, NP), lambda g: (g, 0, 0)),
            _full((8, C)), _full((8, C)), _full((8, C)), _full((C, C)),
            _full((8, C)), _full((C, C)), _full((8, C)), _full((8, C)),
        ],
        out_specs=[
            pl.BlockSpec((1, NP, K), lambda g: (g, 0, 0)),
            pl.BlockSpec((1, NP, 128), lambda g: (g, 0, 0)),
            pl.BlockSpec((1, NP, C), lambda g: (g, 0, 0)),
        ],
        out_shape=[
            jax.ShapeDtypeStruct((B, NP, K), jnp.int32),
            jax.ShapeDtypeStruct((B, NP, 128), jnp.float32),
            jax.ShapeDtypeStruct((B, NP, C), jnp.float32),
        ],
        scratch_shapes=[
            pltpu.VMEM((NP, C), jnp.float32),
            pltpu.VMEM((NP, C), jnp.float32),
            pltpu.VMEM((NP, C), jnp.float32),
        ],
    )(posP, posT, wa1, wp1, _padrow(b1a[None, :]), W1b,
      _padrow(b1b[None, :]), wh2, wp2, _padrow(b2a[None, :]))

    edges = _sc_gather(a2.reshape(B * NP, 128), idxG.reshape(E))

    out = pl.pallas_call(
        _tc2_body,
        grid=(B,),
        in_specs=[
            pl.BlockSpec((1, NP * K, 128), lambda g: (g, 0, 0)),
            pl.BlockSpec((1, NP, C), lambda g: (g, 0, 0)),
            _full((C, C)), _full((8, C)), _full((C, NCLS)),
            _full((8, NCLS)),
        ],
        out_specs=pl.BlockSpec((1, 1, NCLS), lambda g: (g, 0, 0)),
        out_shape=jax.ShapeDtypeStruct((B, 1, NCLS), jnp.float32),
    )(edges.reshape(B, NP * K, 128), b2t, W2b, _padrow(b2b[None, :]), Wc,
      _padrow(bc[None, :]))
    return out.reshape(B, NCLS)


# speedup vs baseline: 1.1031x; 1.0078x over previous
import functools

import jax
import jax.numpy as jnp
from jax import lax
from jax.experimental import pallas as pl
from jax.experimental.pallas import tpu as pltpu
from jax.experimental.pallas import tpu_sc as plsc

B = 10
N = 1000
NP = 1024
K = 16
C = 32
NCLS = 40
TOPK_CHUNK = 128
E = B * NP * K
_BIG = 3.0e38


def _tc1_body(posP, posT, wa1, wp1, b1a, w1b, b1b, wh2, wp2, b2a,
              idxO, aO, bO, aS, bS, h1S):
    g = pl.program_id(0)
    p = posP[0]
    pT = posT[0]
    col_row = lax.broadcasted_iota(jnp.int32, (1, NP), 1)
    sq_row = (jnp.sum(pT * pT, axis=0, keepdims=True)
              + jnp.where(col_row >= N, _BIG, 0.0))
    col_i = lax.broadcasted_iota(jnp.int32, (TOPK_CHUNK, NP), 1)

    aS[:, :] = jnp.dot(p, wa1[:, :]) + b1a[0:1, :]
    bS[:, :] = jnp.dot(p, wp1[:, :])

    def topk_chunk(c, _):
        base = c * TOPK_CHUNK
        pc = posP[0, pl.ds(base, TOPK_CHUNK), :]
        sqc = jnp.sum(pc * pc, axis=1, keepdims=True)
        d = (sqc + sq_row) - 2.0 * jnp.dot(pc, pT)
        bc_ = bS[pl.ds(base, TOPK_CHUNK), :]
        bits = lax.bitcast_convert_type(d, jnp.int32)
        s = bits ^ ((bits >> 31) & jnp.int32(0x7FFFFFFF))
        key = (s & jnp.int32(-1024)) | col_i
        cols = []
        mx = None
        for _k in range(K):
            kmin = jnp.min(key, axis=1, keepdims=True)
            z = key == kmin
            key = jnp.where(z, jnp.int32(0x7FFFFFFF), key)
            cols.append(kmin & jnp.int32(1023))
            zf = z.astype(jnp.float32)
            gk = jnp.dot(zf, aS[:, :])
            msg = jnp.dot(jnp.maximum(gk - bc_, 0.0), w1b[:, :])
            mx = msg if mx is None else jnp.maximum(mx, msg)
        idxO[0, pl.ds(base, TOPK_CHUNK), :] = (
            jnp.concatenate(cols, axis=1) + g * NP)
        h1S[pl.ds(base, TOPK_CHUNK), :] = jnp.maximum(mx + b1b[0:1, :], 0.0)
        return 0

    lax.fori_loop(0, NP // TOPK_CHUNK, topk_chunk, 0, unroll=False)

    p2 = jnp.dot(p, wp2[:, :])
    aO[0, :, 0:C] = jnp.dot(h1S[:, :], wh2[:, :]) + p2 + b2a[0:1, :]
    aO[0, :, C:128] = jnp.zeros((NP, 128 - C), jnp.float32)
    bO[0] = p2


def _tc2_body(edges, b2t, w2b, b2b, wc, bc, out):
    e3 = edges[0, :, 0:C].reshape(NP, K, C)
    pre = e3 - b2t[0][:, None, :]
    act = jnp.maximum(pre, 0.0).reshape(NP * K, C)
    msg = (jnp.dot(act, w2b[:, :]) + b2b[0:1, :]).reshape(NP, K, C)
    h2 = jnp.maximum(jnp.max(msg, axis=1), 0.0)
    row_i = lax.broadcasted_iota(jnp.int32, (NP, C), 0)
    hm = jnp.where(row_i >= N, 0.0, h2)
    gmax = jnp.max(hm, axis=0, keepdims=True)
    out[0] = jnp.dot(gmax, wc[:, :]) + bc[0:1, :]


_SC_CHUNK = 128
_NBUF = 5


def _sc_gather(table, idx):
    info = plsc.get_sparse_core_info()
    nw = info.num_cores * info.num_subcores
    b_per_w = E // nw
    n_chunks = b_per_w // _SC_CHUNK
    mesh = plsc.VectorSubcoreMesh(core_axis_name="c", subcore_axis_name="s")

    @functools.partial(
        pl.kernel, mesh=mesh,
        out_type=jax.ShapeDtypeStruct((E, 128), jnp.float32),
        scratch_types=(
            [pltpu.VMEM((_SC_CHUNK,), jnp.int32)] * _NBUF
            + [pltpu.VMEM((_SC_CHUNK, 128), jnp.float32)] * _NBUF
            + [pltpu.SemaphoreType.DMA] * _NBUF
        ),
    )
    def k(table_hbm, idx_hbm, out_hbm, *bufs):
        idxs = bufs[0:_NBUF]
        rows = bufs[_NBUF:2 * _NBUF]
        sems = bufs[2 * _NBUF:3 * _NBUF]
        wid = lax.axis_index("s") * info.num_cores + lax.axis_index("c")
        base_w = wid * b_per_w

        def chunkn(c, _):
            cps = []
            for i in range(_NBUF):
                bi = base_w + (_NBUF * c + i) * _SC_CHUNK
                pltpu.sync_copy(idx_hbm.at[pl.ds(bi, _SC_CHUNK)], idxs[i])
                cps.append(pltpu.async_copy(
                    table_hbm.at[idxs[i]], rows[i], sems[i]))
            for i in range(_NBUF):
                bi = base_w + (_NBUF * c + i) * _SC_CHUNK
                cps[i].wait()
                pltpu.sync_copy(rows[i], out_hbm.at[pl.ds(bi, _SC_CHUNK)])
            return 0

        lax.fori_loop(0, n_chunks // _NBUF, chunkn, 0, unroll=False)

    return k(table, idx)


def _full(shape):
    nd = len(shape)
    return pl.BlockSpec(shape, lambda g, _n=nd: (0,) * _n)


def _padrow(x, rows=8):
    return jnp.zeros((rows, x.shape[-1]), x.dtype).at[: x.shape[0]].set(x)


@functools.partial(jax.jit, static_argnames=())
def kernel(pos, batch, W1a, b1a, W1b, b1b, W2a, b2a, W2b, b2b, Wc, bc):
    P = pos.reshape(B, N, 3)
    posP = jnp.zeros((B, NP, 8), jnp.float32).at[:, :N, :3].set(P)
    posT = posP.transpose(0, 2, 1)
    wa1 = _padrow(W1a[0:3] + W1a[3:6])
    wp1 = _padrow(W1a[3:6])
    wh2 = W2a[0:C]
    wp2 = _padrow(W2a[C:C + 3])

    idxG, a2, b2t = pl.pallas_call(
        _tc1_body,
        grid=(B,),
        in_specs=[
            pl.BlockSpec((1, NP, 8), lambda g: (g, 0, 0)),
            pl.BlockSpec((1, 8, NP), lambda g: (g, 0, 0)),
            _full((8, C)), _full((8, C)), _full((8, C)), _full((C, C)),
            _full((8, C)), _full((C, C)), _full((8, C)), _full((8, C)),
        ],
        out_specs=[
            pl.BlockSpec((1, NP, K), lambda g: (g, 0, 0)),
            pl.BlockSpec((1, NP, 128), lambda g: (g, 0, 0)),
            pl.BlockSpec((1, NP, C), lambda g: (g, 0, 0)),
        ],
        out_shape=[
            jax.ShapeDtypeStruct((B, NP, K), jnp.int32),
            jax.ShapeDtypeStruct((B, NP, 128), jnp.float32),
            jax.ShapeDtypeStruct((B, NP, C), jnp.float32),
        ],
        scratch_shapes=[
            pltpu.VMEM((NP, C), jnp.float32),
            pltpu.VMEM((NP, C), jnp.float32),
            pltpu.VMEM((NP, C), jnp.float32),
        ],
    )(posP, posT, wa1, wp1, _padrow(b1a[None, :]), W1b,
      _padrow(b1b[None, :]), wh2, wp2, _padrow(b2a[None, :]))

    edges = _sc_gather(a2.reshape(B * NP, 128), idxG.reshape(E))

    out = pl.pallas_call(
        _tc2_body,
        grid=(B,),
        in_specs=[
            pl.BlockSpec((1, NP * K, 128), lambda g: (g, 0, 0)),
            pl.BlockSpec((1, NP, C), lambda g: (g, 0, 0)),
            _full((C, C)), _full((8, C)), _full((C, NCLS)),
            _full((8, NCLS)),
        ],
        out_specs=pl.BlockSpec((1, 1, NCLS), lambda g: (g, 0, 0)),
        out_shape=jax.ShapeDtypeStruct((B, 1, NCLS), jnp.float32),
    )(edges.reshape(B, NP * K, 128), b2t, W2b, _padrow(b2b[None, :]), Wc,
      _padrow(bc[None, :]))
    return out.reshape(B, NCLS)
